# baseline (speedup 1.0000x reference)
import jax
import jax.numpy as jnp
from jax import lax
from jax.experimental import pallas as pl
from jax.experimental.pallas import tpu as pltpu
from jax.experimental.pallas import tpu_sc as plsc

_EMB = 5
_NN = 8
_NE = 16
_NEG = 0.01
_L = 16
_IN_DIMS = [1, 5, 5]


def _sig(x):
    return 1.0 / (1.0 + jnp.exp(-x))


def _lrelu(x):
    return jnp.where(x >= 0, x, _NEG * x)


def _pad16(n):
    return (n + 15) & ~15


class _Packer:
    def __init__(self):
        self.segs = []
        self.offs = {}
        self.pos = 0

    def add(self, name, arr):
        arr = arr.reshape(-1).astype(jnp.float32)
        self.offs[name] = self.pos
        self.segs.append(arr)
        n = arr.shape[0]
        padded = _pad16(n)
        if padded > n:
            self.segs.append(jnp.zeros((padded - n,), jnp.float32))
        self.pos += padded

    def concat(self):
        return jnp.concatenate(self.segs)


def kernel(data, edge_index, d, fw_params, bw_params, find_params):
    ei = edge_index.astype(jnp.int32)

    pg = _Packer()
    pg.add("data", data)
    pg.add("d", d)
    for li, params in enumerate(list(fw_params) + list(bw_params)):
        ws, wm, wg, b = params
        pg.add(f"ws{li}", ws)
        pg.add(f"wm{li}", wm)
        pg.add(f"wg{li}", wg)
        pg.add(f"b{li}", b)
    PG = pg.concat()
    og = pg.offs

    pm = _Packer()
    for li, (W, b) in enumerate(find_params[:3]):
        pm.add(f"W{li}", W)
        pm.add(f"Wb{li}", b)
    W4, b4 = find_params[3]
    pm.add("W4T", W4.T)
    pm.add("b4", b4)
    PM = pm.concat()
    om = pm.offs

    mesh = plsc.VectorSubcoreMesh(core_axis_name="c", subcore_axis_name="s",
                                  num_cores=1, num_subcores=1)

    scratch = [
        pltpu.VMEM((PG.shape[0],), jnp.float32),
        pltpu.VMEM((PM.shape[0],), jnp.float32),
        pltpu.VMEM((2, _L), jnp.int32),
        pltpu.VMEM((_EMB, _L), jnp.float32),
        pltpu.VMEM((_EMB, _L), jnp.float32),
        pltpu.VMEM((_EMB * _L,), jnp.float32),
        pltpu.VMEM((112,), jnp.float32),
        pltpu.VMEM((128,), jnp.float32),
        pltpu.VMEM((128,), jnp.float32),
        pltpu.VMEM((64,), jnp.float32),
        pltpu.VMEM((_L,), jnp.float32),
        pltpu.SemaphoreType.DMA,
        pltpu.SemaphoreType.DMA,
    ]

    def body(pg_hbm, pm_hbm, e_hbm, out_ref, pg_ref, pm_ref, e_ref, x_ref,
             xx_ref, agg_ref, v_ref, h1_ref, h2_ref, h3_ref, outv_ref,
             sem_g, sem_m):
        @pl.when(lax.axis_index("c") == 0)
        def _():
            iota = lax.iota(jnp.int32, _L)
            zero = jnp.zeros((_L,), jnp.float32)

            hm = pltpu.async_copy(pm_hbm, pm_ref, sem_m)
            hg = pltpu.async_copy(pg_hbm, pg_ref, sem_g)
            he = pltpu.async_copy(e_hbm, e_ref, sem_g)
            hg.wait()
            he.wait()

            def rows_of(ref, off, count):
                n_rows = (count + _L - 1) // _L
                return [ref[pl.ds(off + r * _L, _L)] for r in range(n_rows)]

            def splat(rows, i):
                return rows[i // _L].at[
                    jnp.full((_L,), i % _L, jnp.int32)].get(
                        mode="promise_in_bounds")

            src = e_ref[0, :]
            dst = e_ref[1, :]

            x0 = jnp.where(
                iota < _NN,
                plsc.load_gather(pg_ref, [og["data"] + (iota & (_NN - 1))]),
                0.0)
            x_ref[0, :] = x0
            xx_ref[0, :] = x0
            for f in range(1, _EMB):
                x_ref[f, :] = zero
                xx_ref[f, :] = zero

            def gated(xr, li, s_vec, t_vec):
                in_dim = _IN_DIMS[li % 3]
                ws_r = rows_of(pg_ref, og[f"ws{li}"], in_dim * _EMB)
                wm_r = rows_of(pg_ref, og[f"wm{li}"], in_dim * _EMB)
                wg_r = rows_of(pg_ref, og[f"wg{li}"], in_dim)
                b_r = rows_of(pg_ref, og[f"b{li}"], _EMB)
                g = [plsc.load_gather(xr, [jnp.full((_L,), f, jnp.int32), s_vec])
                     for f in range(in_dim)]
                glin = g[0] * splat(wg_r, 0)
                for f in range(1, in_dim):
                    glin = glin + g[f] * splat(wg_r, f)
                gate = _sig(glin)
                for k in range(_EMB):
                    agg_ref[pl.ds(k * _L, _L)] = zero
                for k in range(_EMB):
                    msg = g[0] * splat(wm_r, k)
                    for f in range(1, in_dim):
                        msg = msg + g[f] * splat(wm_r, f * _EMB + k)
                    plsc.addupdate_scatter(agg_ref, [t_vec + k * _L],
                                           gate * msg)
                xs = [xr[f, :] for f in range(in_dim)]
                new = []
                for k in range(_EMB):
                    acc = agg_ref[pl.ds(k * _L, _L)] + splat(b_r, k)
                    for f in range(in_dim):
                        acc = acc + xs[f] * splat(ws_r, f * _EMB + k)
                    new.append(_lrelu(acc))
                for k in range(_EMB):
                    xr[k, :] = new[k]

            for l in range(3):
                gated(x_ref, l, src, dst)
            for l in range(3):
                gated(xx_ref, 3 + l, dst, src)

            lane_mask = iota < _NN
            for f in range(_EMB):
                plsc.store_scatter(v_ref, [iota * _EMB + f], x_ref[f, :],
                                   mask=lane_mask)
                plsc.store_scatter(v_ref, [40 + iota * _EMB + f], xx_ref[f, :],
                                   mask=lane_mask)
            d_o = og["d"]
            v_ref[pl.ds(80, _L)] = pg_ref[pl.ds(d_o, _L)]
            v_ref[pl.ds(96, _L)] = jnp.where(
                iota < 8,
                plsc.load_gather(pg_ref, [d_o + jnp.minimum(iota + 16, 23)]),
                0.0)

            hm.wait()

            def dense(src_ref, w_o, b_o, in_dim, out_dim, dst_ref, act):
                nchunk = out_dim // _L
                accs = [pm_ref[pl.ds(b_o + _L * c, _L)] for c in range(nchunk)]
                for blk in range(0, in_dim, _L):
                    row = src_ref[pl.ds(blk, _L)]
                    for lane in range(min(_L, in_dim - blk)):
                        i = blk + lane
                        bv = row.at[jnp.full((_L,), lane, jnp.int32)].get(
                            mode="promise_in_bounds")
                        for c in range(nchunk):
                            accs[c] = accs[c] + bv * pm_ref[
                                pl.ds(w_o + i * out_dim + _L * c, _L)]
                for c in range(nchunk):
                    dst_ref[pl.ds(_L * c, _L)] = act(accs[c])

            dense(v_ref, om["W0"], om["Wb0"], 104, 128, h1_ref, _lrelu)
            dense(h1_ref, om["W1"], om["Wb1"], 128, 128, h2_ref, _lrelu)
            dense(h2_ref, om["W2"], om["Wb2"], 128, 64, h3_ref, _lrelu)

            b4_r = rows_of(pm_ref, om["b4"], 2)
            outs = []
            for j in range(2):
                t = zero
                for q in range(4):
                    t = t + (h3_ref[pl.ds(_L * q, _L)]
                             * pm_ref[pl.ds(om["W4T"] + j * 64 + _L * q, _L)])
                tj = jnp.sum(t)
                outs.append(_sig(jnp.full((_L,), tj) + splat(b4_r, j)))
            outv_ref[...] = 0.5 * outs[0] + 0.5 * outs[1]
            pltpu.sync_copy(outv_ref.at[pl.ds(0, _NN)], out_ref)

    run = pl.kernel(
        body,
        out_type=jax.ShapeDtypeStruct((_NN,), jnp.float32),
        mesh=mesh,
        scratch_types=scratch,
        compiler_params=pltpu.CompilerParams(needs_layout_passes=False),
    )
    out = run(PG, PM, ei)
    return out[0]


# speedup vs baseline: 1.0267x; 1.0267x over previous
import jax
import jax.numpy as jnp
from jax import lax
from jax.experimental import pallas as pl
from jax.experimental.pallas import tpu as pltpu
from jax.experimental.pallas import tpu_sc as plsc

_EMB = 5
_NN = 8
_NE = 16
_NEG = 0.01
_L = 16
_IN_DIMS = [1, 5, 5]

_W1S, _B1S = 0, 1664
_W2S, _B2S = 1680, 3728
_W3S, _B3S = 3744, 5792
_W4TS, _B4S = 5808, 5936
_DS = 5952
_SEC = 5984


def _sig(x):
    return 1.0 / (1.0 + jnp.exp(-x))


def _lrelu(x):
    return jnp.where(x >= 0, x, _NEG * x)


def _pad16(n):
    return (n + 15) & ~15


class _Packer:
    def __init__(self):
        self.segs = []
        self.offs = {}
        self.pos = 0

    def add(self, name, arr):
        arr = arr.reshape(-1).astype(jnp.float32)
        self.offs[name] = self.pos
        self.segs.append(arr)
        n = arr.shape[0]
        padded = _pad16(n)
        if padded > n:
            self.segs.append(jnp.zeros((padded - n,), jnp.float32))
        self.pos += padded

    def concat(self):
        return jnp.concatenate(self.segs)


def kernel(data, edge_index, d, fw_params, bw_params, find_params):
    ei = edge_index.astype(jnp.int32)

    pg = _Packer()
    pg.add("data", data)
    for li, params in enumerate(list(fw_params) + list(bw_params)):
        ws, wm, wg, b = params
        pg.add(f"ws{li}", ws)
        pg.add(f"wm{li}", wm)
        pg.add(f"wg{li}", wg)
        pg.add(f"b{li}", b)
    PG = pg.concat()
    og = pg.offs

    (W1, b1), (W2, b2), (W3, b3), (W4, b4) = find_params
    d_flat = d.reshape(-1).astype(jnp.float32)
    zcol16 = jnp.zeros((128, 16), jnp.float32)
    secs = []
    for t in range(8):
        parts = [
            W1[:, 16 * t:16 * t + 16].reshape(-1),
            b1[16 * t:16 * t + 16],
            W2[:, 16 * t:16 * t + 16].reshape(-1),
            b2[16 * t:16 * t + 16],
            (W3[:, 16 * t:16 * t + 16] if t < 4 else zcol16).reshape(-1),
            (b3[16 * t:16 * t + 16] if t < 4 else jnp.zeros((16,), jnp.float32)),
            (W4.T.reshape(-1) if t == 0 else jnp.zeros((128,), jnp.float32)),
            (jnp.concatenate([b4, jnp.zeros((14,), jnp.float32)])
             if t == 0 else jnp.zeros((16,), jnp.float32)),
            d_flat,
            jnp.zeros((8,), jnp.float32),
        ]
        secs.append(jnp.concatenate([p.astype(jnp.float32) for p in parts]))
    PM = jnp.concatenate(secs)

    mesh = plsc.VectorSubcoreMesh(core_axis_name="c", subcore_axis_name="s",
                                  num_cores=1)

    scratch = [
        pltpu.VMEM((PG.shape[0],), jnp.float32),
        pltpu.VMEM((_SEC,), jnp.float32),
        pltpu.VMEM((2, _L), jnp.int32),
        pltpu.VMEM((_EMB * _L,), jnp.float32),
        pltpu.VMEM((_EMB * _L,), jnp.float32),
        pltpu.VMEM((2 * _EMB * _L,), jnp.float32),
        pltpu.VMEM((112,), jnp.float32),
        pltpu.VMEM((128,), jnp.float32),
        pltpu.VMEM((64,), jnp.float32),
        pltpu.VMEM((_L,), jnp.float32),
        pltpu.VMEM_SHARED((2 * _EMB * _L,), jnp.float32),
        pltpu.VMEM_SHARED((128,), jnp.float32),
        pltpu.VMEM_SHARED((128,), jnp.float32),
        pltpu.VMEM_SHARED((64,), jnp.float32),
        pltpu.SemaphoreType.DMA,
        pltpu.SemaphoreType.DMA,
    ]

    def body(pg_hbm, pm_hbm, e_hbm, out_ref, pg_ref, pms_ref, e_ref, x_ref,
             agg_ref, xs_loc, v_ref, h_loc, h3_loc, stage_ref,
             xs_sh, h1_sh, h2_sh, h3_sh, sem_g, sem_m):
        sid = lax.axis_index("s")
        iota = lax.iota(jnp.int32, _L)
        zero = jnp.zeros((_L,), jnp.float32)

        def rows_of(ref, off, count):
            n_rows = (count + _L - 1) // _L
            return [ref[pl.ds(off + r * _L, _L)] for r in range(n_rows)]

        def splat(rows, i):
            return rows[i // _L].at[
                jnp.full((_L,), i % _L, jnp.int32)].get(
                    mode="promise_in_bounds")

        @pl.when(sid < 8)
        def _():
            pltpu.async_copy(pm_hbm.at[pl.ds(sid * _SEC, _SEC)], pms_ref,
                             sem_m).wait()

        @pl.when(sid < 2)
        def _():
            hg = pltpu.async_copy(pg_hbm, pg_ref, sem_g)
            he = pltpu.async_copy(e_hbm, e_ref, sem_g)
            hg.wait()
            he.wait()

            src = e_ref[0, :]
            dst = e_ref[1, :]
            x0 = jnp.where(
                iota < _NN,
                plsc.load_gather(pg_ref, [og["data"] + (iota & (_NN - 1))]),
                0.0)
            x_ref[pl.ds(0, _L)] = x0
            for f in range(1, _EMB):
                x_ref[pl.ds(f * _L, _L)] = zero

            def gated(li, s_vec, t_vec):
                in_dim = _IN_DIMS[li % 3]
                ws_r = rows_of(pg_ref, og[f"ws{li}"], in_dim * _EMB)
                wm_r = rows_of(pg_ref, og[f"wm{li}"], in_dim * _EMB)
                wg_r = rows_of(pg_ref, og[f"wg{li}"], in_dim)
                b_r = rows_of(pg_ref, og[f"b{li}"], _EMB)
                g = [plsc.load_gather(x_ref, [s_vec + f * _L])
                     for f in range(in_dim)]
                glin = g[0] * splat(wg_r, 0)
                for f in range(1, in_dim):
                    glin = glin + g[f] * splat(wg_r, f)
                gate = _sig(glin)
                for k in range(_EMB):
                    agg_ref[pl.ds(k * _L, _L)] = zero
                for k in range(_EMB):
                    msg = g[0] * splat(wm_r, k)
                    for f in range(1, in_dim):
                        msg = msg + g[f] * splat(wm_r, f * _EMB + k)
                    plsc.addupdate_scatter(agg_ref, [t_vec + k * _L],
                                           gate * msg)
                xs = [x_ref[pl.ds(f * _L, _L)] for f in range(in_dim)]
                new = []
                for k in range(_EMB):
                    acc = agg_ref[pl.ds(k * _L, _L)] + splat(b_r, k)
                    for f in range(in_dim):
                        acc = acc + xs[f] * splat(ws_r, f * _EMB + k)
                    new.append(_lrelu(acc))
                for k in range(_EMB):
                    x_ref[pl.ds(k * _L, _L)] = new[k]

            @pl.when(sid == 0)
            def _():
                for l in range(3):
                    gated(l, src, dst)

            @pl.when(sid == 1)
            def _():
                for l in range(3):
                    gated(3 + l, dst, src)

            pltpu.sync_copy(x_ref, xs_sh.at[pl.ds(sid * _EMB * _L,
                                                  _EMB * _L)])

        plsc.subcore_barrier()

        @pl.when(sid < 8)
        def _():
            pltpu.sync_copy(xs_sh, xs_loc)
            lane_mask = iota < _NN
            for f in range(_EMB):
                plsc.store_scatter(v_ref, [iota * _EMB + f],
                                   xs_loc[pl.ds(f * _L, _L)], mask=lane_mask)
                plsc.store_scatter(v_ref, [40 + iota * _EMB + f],
                                   xs_loc[pl.ds((_EMB + f) * _L, _L)],
                                   mask=lane_mask)
            v_ref[pl.ds(80, _L)] = pms_ref[pl.ds(_DS, _L)]
            v_ref[pl.ds(96, _L)] = jnp.where(
                iota < 8,
                plsc.load_gather(pms_ref, [_DS + jnp.minimum(iota + 16, 23)]),
                0.0)

        def dense_chunk(src_ref, w_o, b_o, in_dim, act):
            acc = pms_ref[pl.ds(b_o, _L)]
            for blk in range(0, in_dim, _L):
                row = src_ref[pl.ds(blk, _L)]
                for lane in range(min(_L, in_dim - blk)):
                    i = blk + lane
                    bv = row.at[jnp.full((_L,), lane, jnp.int32)].get(
                        mode="promise_in_bounds")
                    acc = acc + bv * pms_ref[pl.ds(w_o + i * _L, _L)]
            return act(acc)

        @pl.when(sid < 8)
        def _():
            stage_ref[...] = dense_chunk(v_ref, _W1S, _B1S, 104, _lrelu)
            pltpu.sync_copy(stage_ref, h1_sh.at[pl.ds(sid * _L, _L)])

        plsc.subcore_barrier()

        @pl.when(sid < 8)
        def _():
            pltpu.sync_copy(h1_sh, h_loc)
            stage_ref[...] = dense_chunk(h_loc, _W2S, _B2S, 128, _lrelu)
            pltpu.sync_copy(stage_ref, h2_sh.at[pl.ds(sid * _L, _L)])

        plsc.subcore_barrier()

        @pl.when(sid < 4)
        def _():
            pltpu.sync_copy(h2_sh, h_loc)
            stage_ref[...] = dense_chunk(h_loc, _W3S, _B3S, 128, _lrelu)
            pltpu.sync_copy(stage_ref, h3_sh.at[pl.ds(sid * _L, _L)])

        plsc.subcore_barrier()

        @pl.when(sid == 0)
        def _():
            pltpu.sync_copy(h3_sh, h3_loc)
            b4_r = rows_of(pms_ref, _B4S, 2)
            outs = []
            for j in range(2):
                t = zero
                for q in range(4):
                    t = t + (h3_loc[pl.ds(_L * q, _L)]
                             * pms_ref[pl.ds(_W4TS + j * 64 + _L * q, _L)])
                tj = jnp.sum(t)
                outs.append(_sig(jnp.full((_L,), tj) + splat(b4_r, j)))
            stage_ref[...] = 0.5 * outs[0] + 0.5 * outs[1]
            pltpu.sync_copy(stage_ref.at[pl.ds(0, _NN)], out_ref)

    run = pl.kernel(
        body,
        out_type=jax.ShapeDtypeStruct((_NN,), jnp.float32),
        mesh=mesh,
        scratch_types=scratch,
        compiler_params=pltpu.CompilerParams(needs_layout_passes=False),
    )
    out = run(PG, PM, ei)
    return out[0]
